# Initial kernel scaffold; baseline (speedup 1.0000x reference)
import jax
import jax.numpy as jnp
from jax.experimental import pallas as pl


def kernel(x, W1, b1, W2, b2):
    raise NotImplementedError("write your pallas kernel here")


# speedup vs baseline: 3.4712x; 3.4712x over previous
import functools
import math

import jax
import jax.numpy as jnp
from jax.experimental import pallas as pl
from jax.experimental.pallas import tpu as pltpu


def _gelu(x):
    c = math.sqrt(2.0 / math.pi)
    return 0.5 * x * (1.0 + jnp.tanh(c * (x + 0.044715 * x * x * x)))


def _pick(total, candidates):
    for c in candidates:
        if total >= c and total % c == 0:
            return c
    return total


def _ffn_kernel(x_ref, w1_ref, b1_ref, w2_ref, b2_ref, o_ref, h_ref, *, tn):
    xb = x_ref[...].astype(jnp.bfloat16)
    inner = w1_ref.shape[1]
    for j in range(inner // tn):
        sl = slice(j * tn, (j + 1) * tn)
        z = jnp.dot(xb, w1_ref[:, sl], preferred_element_type=jnp.float32)
        h_ref[:, sl] = _gelu(z + b1_ref[:, sl]).astype(jnp.bfloat16)
    acc = jnp.dot(h_ref[...], w2_ref[...], preferred_element_type=jnp.float32)
    o_ref[...] = acc + b2_ref[...]


def kernel(x, W1, b1, W2, b2):
    lead = x.shape[:-1]
    Din = x.shape[-1]
    inner = W1.shape[1]
    Dout = W2.shape[1]

    x2d = x.reshape(-1, Din)
    T0 = x2d.shape[0]
    pad = (-T0) % 8
    if pad:
        x2d = jnp.pad(x2d, ((0, pad), (0, 0)))
    T = x2d.shape[0]

    tm = _pick(T, (512, 256, 128, 64, 32, 16, 8))
    tn = _pick(inner, (512, 256, 128))

    w1b = W1.astype(jnp.bfloat16)
    w2b = W2.astype(jnp.bfloat16)
    b1r = b1.reshape(1, inner)
    b2r = b2.reshape(1, Dout)

    out = pl.pallas_call(
        functools.partial(_ffn_kernel, tn=tn),
        out_shape=jax.ShapeDtypeStruct((T, Dout), x.dtype),
        grid=(T // tm,),
        in_specs=[
            pl.BlockSpec((tm, Din), lambda i: (i, 0)),
            pl.BlockSpec((Din, inner), lambda i: (0, 0)),
            pl.BlockSpec((1, inner), lambda i: (0, 0)),
            pl.BlockSpec((inner, Dout), lambda i: (0, 0)),
            pl.BlockSpec((1, Dout), lambda i: (0, 0)),
        ],
        out_specs=pl.BlockSpec((tm, Dout), lambda i: (i, 0)),
        scratch_shapes=[pltpu.VMEM((tm, inner), jnp.bfloat16)],
        compiler_params=pltpu.CompilerParams(
            dimension_semantics=("parallel",)),
    )(x2d, w1b, b1r, w2b, b2r)

    if T != T0:
        out = out[:T0]
    return out.reshape(*lead, Dout)
